# Initial kernel scaffold; baseline (speedup 1.0000x reference)
import jax
import jax.numpy as jnp
from jax.experimental import pallas as pl


def kernel(in_feat, edge_index, W1, al1, ar1, b1, W2, al2, ar2, b2, W3, al3, ar3, b3):
    raise NotImplementedError("write your pallas kernel here")


# speedup vs baseline: 7.5311x; 7.5311x over previous
import functools

import jax
import jax.numpy as jnp
from jax import lax
from jax.experimental import pallas as pl
from jax.experimental.pallas import tpu as pltpu
from jax.experimental.pallas import tpu_sc as plsc

N = 10000
E = 320000
D = 128

NC = 2
NS = 16
L = 16
NW = NC * NS

C = 128
EPW = 10240
NCH = EPW // C
EPAD = NW * EPW
NPAD = 10240
ROWS_PW = NPAD // NS

_mesh = plsc.VectorSubcoreMesh(
    core_axis_name="c", subcore_axis_name="s", num_cores=NC, num_subcores=NS
)


def _edge_weights(el_v, er_v, srcb, dstb, i):
    s16 = srcb[pl.ds(i * L, L)]
    d16 = dstb[pl.ds(i * L, L)]
    elg = plsc.load_gather(el_v, [s16])
    erg = plsc.load_gather(er_v, [d16])
    x = elg + erg
    e = jnp.where(x >= 0.0, x, 0.2 * x)
    return jnp.exp(e), d16


def _sc_denom_body(src_hbm, dst_hbm, el_hbm, er_hbm, den_out,
                   el_v, er_v, srcb, dstb, exb, zb, den_sh):
    cid = lax.axis_index("c")
    sid = lax.axis_index("s")
    wid = sid * NC + cid
    tbase = sid * ROWS_PW

    def zinit(i, _):
        zb[pl.ds(i * L, L)] = jnp.zeros((L,), jnp.float32)
        return 0

    lax.fori_loop(0, ROWS_PW // L, zinit, 0)
    pltpu.sync_copy(zb, den_sh.at[pl.ds(tbase, ROWS_PW)])
    pltpu.sync_copy(el_hbm, el_v)
    pltpu.sync_copy(er_hbm, er_v)
    plsc.subcore_barrier()

    ebase = wid * EPW

    def chunk(j, _):
        eb = ebase + j * C
        pltpu.sync_copy(src_hbm.at[pl.ds(eb, C)], srcb)
        pltpu.sync_copy(dst_hbm.at[pl.ds(eb, C)], dstb)
        for i in range(C // L):
            ex, _d = _edge_weights(el_v, er_v, srcb, dstb, i)
            exb[pl.ds(i * L, L)] = ex
        pltpu.sync_copy(exb, den_sh.at[dstb], add=True)
        return 0

    lax.fori_loop(0, NCH, chunk, 0)
    plsc.subcore_barrier()
    pltpu.sync_copy(den_sh.at[pl.ds(tbase, ROWS_PW)],
                    den_out.at[cid, pl.ds(tbase, ROWS_PW)])


_sc_denom = functools.partial(
    pl.kernel,
    out_type=jax.ShapeDtypeStruct((NC, NPAD), jnp.float32),
    mesh=_mesh,
    compiler_params=pltpu.CompilerParams(needs_layout_passes=False, use_tc_tiling_on_sc=False),
    scratch_types=[
        pltpu.VMEM((NPAD,), jnp.float32),
        pltpu.VMEM((NPAD,), jnp.float32),
        pltpu.VMEM((C,), jnp.int32),
        pltpu.VMEM((C,), jnp.int32),
        pltpu.VMEM((C,), jnp.float32),
        pltpu.VMEM((ROWS_PW,), jnp.float32),
        pltpu.VMEM_SHARED((NPAD,), jnp.float32),
    ],
)(_sc_denom_body)


def _sc_alpha_body(src_hbm, dst_hbm, el_hbm, er_hbm, den_hbm, alpha_out,
                   el_v, er_v, den_v, tmp_v, srcb, dstb, alphab):
    cid = lax.axis_index("c")
    sid = lax.axis_index("s")
    wid = sid * NC + cid

    pltpu.sync_copy(el_hbm, el_v)
    pltpu.sync_copy(er_hbm, er_v)
    pltpu.sync_copy(den_hbm.at[0], den_v)
    pltpu.sync_copy(den_hbm.at[1], tmp_v)

    def dsum(i, _):
        sl = pl.ds(i * L, L)
        den_v[sl] = den_v[sl] + tmp_v[sl] + 1e-9
        return 0

    lax.fori_loop(0, NPAD // L, dsum, 0)

    ebase = wid * EPW

    def chunk(j, _):
        eb = ebase + j * C
        pltpu.sync_copy(src_hbm.at[pl.ds(eb, C)], srcb)
        pltpu.sync_copy(dst_hbm.at[pl.ds(eb, C)], dstb)
        for i in range(C // L):
            ex, d16 = _edge_weights(el_v, er_v, srcb, dstb, i)
            dg = plsc.load_gather(den_v, [d16])
            alphab[pl.ds(i * L, L)] = ex / dg
        pltpu.sync_copy(alphab, alpha_out.at[pl.ds(eb, C)])
        return 0

    lax.fori_loop(0, NCH, chunk, 0)


_sc_alpha = functools.partial(
    pl.kernel,
    out_type=jax.ShapeDtypeStruct((EPAD,), jnp.float32),
    mesh=_mesh,
    compiler_params=pltpu.CompilerParams(needs_layout_passes=False, use_tc_tiling_on_sc=False),
    scratch_types=[
        pltpu.VMEM((NPAD,), jnp.float32),
        pltpu.VMEM((NPAD,), jnp.float32),
        pltpu.VMEM((NPAD,), jnp.float32),
        pltpu.VMEM((NPAD,), jnp.float32),
        pltpu.VMEM((C,), jnp.int32),
        pltpu.VMEM((C,), jnp.int32),
        pltpu.VMEM((C,), jnp.float32),
    ],
)(_sc_alpha_body)

DH = D // 2


def _sc_agg_body(src_hbm, dst_hbm, alpha_hbm, feat_hbm, out_hbm,
                 srcb, dstb, alphab, rows, out_sh, sem):
    cid = lax.axis_index("c")
    sid = lax.axis_index("s")
    wid = sid * NC + cid
    tbase = sid * ROWS_PW

    def zrow(r, _):
        for k in range(DH // L):
            rows[r, pl.ds(k * L, L)] = jnp.zeros((L,), jnp.float32)
        return 0

    lax.fori_loop(0, C, zrow, 0)
    for k in range(ROWS_PW // C):
        pltpu.sync_copy(rows, out_sh.at[pl.ds(tbase + k * C, C)])
    plsc.subcore_barrier()

    ebase = wid * EPW

    def chunk(j, _):
        eb = ebase + j * C
        pltpu.sync_copy(src_hbm.at[pl.ds(eb, C)], srcb)
        pltpu.sync_copy(dst_hbm.at[pl.ds(eb, C)], dstb)
        pltpu.sync_copy(alpha_hbm.at[pl.ds(eb, C)], alphab)
        pltpu.async_copy(feat_hbm.at[srcb], rows, sem).wait()

        def scale(g, _):
            a16 = alphab[pl.ds(g * L, L)]
            for j2 in range(L):
                r = g * L + j2
                av = jnp.full((L,), a16[j2], jnp.float32)
                for k in range(DH // L):
                    sl = pl.ds(k * L, L)
                    rows[r, sl] = rows[r, sl] * av
            return 0

        lax.fori_loop(0, C // L, scale, 0)
        pltpu.sync_copy(rows, out_sh.at[dstb], add=True)
        return 0

    lax.fori_loop(0, NCH, chunk, 0)
    plsc.subcore_barrier()
    pltpu.sync_copy(out_sh.at[pl.ds(tbase, ROWS_PW)],
                    out_hbm.at[cid, pl.ds(tbase, ROWS_PW)])


_sc_agg = functools.partial(
    pl.kernel,
    out_type=jax.ShapeDtypeStruct((NC, NPAD, DH), jnp.float32),
    mesh=_mesh,
    compiler_params=pltpu.CompilerParams(needs_layout_passes=False, use_tc_tiling_on_sc=False),
    scratch_types=[
        pltpu.VMEM((C,), jnp.int32),
        pltpu.VMEM((C,), jnp.int32),
        pltpu.VMEM((C,), jnp.float32),
        pltpu.VMEM((C, DH), jnp.float32),
        pltpu.VMEM_SHARED((NPAD, DH), jnp.float32),
        pltpu.SemaphoreType.DMA,
    ],
)(_sc_agg_body)


def _tc_first_body(h_ref, w_ref, al_ref, ar_ref,
                   flo_ref, fhi_ref, el_ref, er_ref):
    f = jnp.dot(h_ref[...], w_ref[...], preferred_element_type=jnp.float32)
    flo_ref[...] = f[:, :DH]
    fhi_ref[...] = f[:, DH:]
    el_ref[...] = jnp.sum(f * al_ref[...][None, :], axis=1)
    er_ref[...] = jnp.sum(f * ar_ref[...][None, :], axis=1)


_tc_out_types = [
    jax.ShapeDtypeStruct((NPAD, DH), jnp.float32),
    jax.ShapeDtypeStruct((NPAD, DH), jnp.float32),
    jax.ShapeDtypeStruct((NPAD,), jnp.float32),
    jax.ShapeDtypeStruct((NPAD,), jnp.float32),
]


def _tc_first(h, w, al, ar):
    return pl.pallas_call(_tc_first_body, out_shape=_tc_out_types)(
        h, w, al, ar)


def _tc_mid_body(plo_ref, phi_ref, b_ref, w_ref, al_ref, ar_ref,
                 flo_ref, fhi_ref, el_ref, er_ref):
    hm = jnp.concatenate(
        [plo_ref[0] + plo_ref[1], phi_ref[0] + phi_ref[1]], axis=1)
    hm = jnp.maximum(hm + b_ref[...][None, :], 0.0)
    f = jnp.dot(hm, w_ref[...], preferred_element_type=jnp.float32)
    flo_ref[...] = f[:, :DH]
    fhi_ref[...] = f[:, DH:]
    el_ref[...] = jnp.sum(f * al_ref[...][None, :], axis=1)
    er_ref[...] = jnp.sum(f * ar_ref[...][None, :], axis=1)


def _tc_mid(plo, phi, b, w, al, ar):
    return pl.pallas_call(_tc_mid_body, out_shape=_tc_out_types)(
        plo, phi, b, w, al, ar)


def _tc_final_body(plo_ref, phi_ref, b_ref, out_ref):
    hm = jnp.concatenate(
        [plo_ref[0] + plo_ref[1], phi_ref[0] + phi_ref[1]], axis=1)
    out_ref[...] = hm + b_ref[...][None, :]


def _tc_final(plo, phi, b):
    return pl.pallas_call(
        _tc_final_body,
        out_shape=jax.ShapeDtypeStruct((NPAD, D), jnp.float32),
    )(plo, phi, b)


def kernel(in_feat, edge_index, W1, al1, ar1, b1, W2, al2, ar2, b2,
           W3, al3, ar3, b3):
    src = jnp.concatenate(
        [edge_index[0], jnp.zeros((EPAD - E,), jnp.int32)])
    dst = jnp.concatenate(
        [edge_index[1], jnp.full((EPAD - E,), N, jnp.int32)])
    h = jnp.pad(in_feat, ((0, NPAD - N), (0, 0)))

    flo, fhi, el, er = _tc_first(h, W1, al1, ar1)
    for (b, w, al, ar) in ((b1, W2, al2, ar2), (b2, W3, al3, ar3)):
        den = _sc_denom(src, dst, el, er)
        alpha = _sc_alpha(src, dst, el, er, den)
        plo = _sc_agg(src, dst, alpha, flo)
        phi = _sc_agg(src, dst, alpha, fhi)
        flo, fhi, el, er = _tc_mid(plo, phi, b, w, al, ar)

    den = _sc_denom(src, dst, el, er)
    alpha = _sc_alpha(src, dst, el, er, den)
    plo = _sc_agg(src, dst, alpha, flo)
    phi = _sc_agg(src, dst, alpha, fhi)
    out = _tc_final(plo, phi, b3)
    return out[:N]


# speedup vs baseline: 14.7168x; 1.9541x over previous
import functools

import jax
import jax.numpy as jnp
from jax import lax
from jax.experimental import pallas as pl
from jax.experimental.pallas import tpu as pltpu
from jax.experimental.pallas import tpu_sc as plsc

N = 10000
E = 320000
D = 128

NC = 2
NS = 16
L = 16
NW = NC * NS

C = 128
EPW = 10240
NCH = EPW // C
EPAD = NW * EPW
NPAD = 10240
ROWS_PW = NPAD // NS

_mesh = plsc.VectorSubcoreMesh(
    core_axis_name="c", subcore_axis_name="s", num_cores=NC, num_subcores=NS
)


def _ew16(el_v, er_v, s16, d16):
    elg = plsc.load_gather(el_v, [s16])
    erg = plsc.load_gather(er_v, [d16])
    x = elg + erg
    e = jnp.where(x >= 0.0, x, 0.2 * x)
    return jnp.exp(e)


_sc_params = pltpu.CompilerParams(
    needs_layout_passes=False, use_tc_tiling_on_sc=False)


def _sc_denom_body(src_hbm, dst_hbm, el_hbm, er_hbm, den_out,
                   el_v, er_v, srcw, dstw, exb, zb, den_sh):
    cid = lax.axis_index("c")
    sid = lax.axis_index("s")
    wid = sid * NC + cid
    tbase = sid * ROWS_PW

    def zinit(i, _):
        zb[pl.ds(i * L, L)] = jnp.zeros((L,), jnp.float32)
        return 0

    lax.fori_loop(0, ROWS_PW // L, zinit, 0)
    pltpu.sync_copy(zb, den_sh.at[pl.ds(tbase, ROWS_PW)])
    pltpu.sync_copy(el_hbm, el_v)
    pltpu.sync_copy(er_hbm, er_v)
    pltpu.sync_copy(src_hbm.at[wid], srcw)
    pltpu.sync_copy(dst_hbm.at[wid], dstw)
    plsc.subcore_barrier()

    def chunk(j, _):
        for i in range(C // L):
            sl = pl.ds(i * L, L)
            exb[sl] = _ew16(el_v, er_v, srcw[j, sl], dstw[j, sl])
        pltpu.sync_copy(exb, den_sh.at[dstw.at[j]], add=True)
        return 0

    lax.fori_loop(0, NCH, chunk, 0)
    plsc.subcore_barrier()
    pltpu.sync_copy(den_sh.at[pl.ds(tbase, ROWS_PW)],
                    den_out.at[cid, pl.ds(tbase, ROWS_PW)])


_sc_denom = functools.partial(
    pl.kernel,
    out_type=jax.ShapeDtypeStruct((NC, NPAD), jnp.float32),
    mesh=_mesh,
    compiler_params=_sc_params,
    scratch_types=[
        pltpu.VMEM((NPAD,), jnp.float32),
        pltpu.VMEM((NPAD,), jnp.float32),
        pltpu.VMEM((NCH, C), jnp.int32),
        pltpu.VMEM((NCH, C), jnp.int32),
        pltpu.VMEM((C,), jnp.float32),
        pltpu.VMEM((ROWS_PW,), jnp.float32),
        pltpu.VMEM_SHARED((NPAD,), jnp.float32),
    ],
)(_sc_denom_body)


def _sc_alpha_body(src_hbm, dst_hbm, el_hbm, er_hbm, den_hbm, alpha_out,
                   el_v, er_v, den_v, tmp_v, srcw, dstw, alphaw):
    cid = lax.axis_index("c")
    sid = lax.axis_index("s")
    wid = sid * NC + cid

    pltpu.sync_copy(el_hbm, el_v)
    pltpu.sync_copy(er_hbm, er_v)
    pltpu.sync_copy(den_hbm.at[0], den_v)
    pltpu.sync_copy(den_hbm.at[1], tmp_v)
    pltpu.sync_copy(src_hbm.at[wid], srcw)
    pltpu.sync_copy(dst_hbm.at[wid], dstw)

    def dsum(i, _):
        sl = pl.ds(i * L, L)
        den_v[sl] = den_v[sl] + tmp_v[sl] + 1e-9
        return 0

    lax.fori_loop(0, NPAD // L, dsum, 0)

    def chunk(j, _):
        for i in range(C // L):
            sl = pl.ds(i * L, L)
            d16 = dstw[j, sl]
            ex = _ew16(el_v, er_v, srcw[j, sl], d16)
            dg = plsc.load_gather(den_v, [d16])
            alphaw[j, sl] = ex / dg
        return 0

    lax.fori_loop(0, NCH, chunk, 0)
    pltpu.sync_copy(alphaw, alpha_out.at[wid])


_sc_alpha = functools.partial(
    pl.kernel,
    out_type=jax.ShapeDtypeStruct((NW, NCH, C), jnp.float32),
    mesh=_mesh,
    compiler_params=_sc_params,
    scratch_types=[
        pltpu.VMEM((NPAD,), jnp.float32),
        pltpu.VMEM((NPAD,), jnp.float32),
        pltpu.VMEM((NPAD,), jnp.float32),
        pltpu.VMEM((NPAD,), jnp.float32),
        pltpu.VMEM((NCH, C), jnp.int32),
        pltpu.VMEM((NCH, C), jnp.int32),
        pltpu.VMEM((NCH, C), jnp.float32),
    ],
)(_sc_alpha_body)

DH = D // 2


def _sc_agg_body(src_hbm, dst_hbm, alpha_hbm, feat_hbm, out_hbm,
                 srcw, dstw, alphaw, rows0, rows1, out_sh, sg0, sg1):
    cid = lax.axis_index("c")
    sid = lax.axis_index("s")
    wid = sid * NC + cid
    tbase = sid * ROWS_PW

    pltpu.sync_copy(src_hbm.at[wid], srcw)
    pltpu.sync_copy(dst_hbm.at[wid], dstw)
    pltpu.sync_copy(alpha_hbm.at[wid], alphaw)

    def zrow(r, _):
        for k in range(DH // L):
            rows0[r, pl.ds(k * L, L)] = jnp.zeros((L,), jnp.float32)
        return 0

    lax.fori_loop(0, C, zrow, 0)
    for k in range(ROWS_PW // C):
        pltpu.sync_copy(rows0, out_sh.at[pl.ds(tbase + k * C, C)])
    plsc.subcore_barrier()

    def scale_scatter(j, rows):
        def scale(g, _):
            a16 = alphaw[j, pl.ds(g * L, L)]
            for j2 in range(L):
                r = g * L + j2
                av = jnp.full((L,), a16[j2], jnp.float32)
                for k in range(DH // L):
                    sl = pl.ds(k * L, L)
                    rows[r, sl] = rows[r, sl] * av
            return 0

        lax.fori_loop(0, C // L, scale, 0)
        pltpu.sync_copy(rows, out_sh.at[dstw.at[j]], add=True)

    pltpu.async_copy(feat_hbm.at[srcw.at[0]], rows0, sg0)

    def pipe(j2, _):
        e = 2 * j2
        o = e + 1
        pltpu.make_async_copy(feat_hbm.at[srcw.at[e]], rows0, sg0).wait()
        pltpu.async_copy(feat_hbm.at[srcw.at[o]], rows1, sg1)
        scale_scatter(e, rows0)
        pltpu.make_async_copy(feat_hbm.at[srcw.at[o]], rows1, sg1).wait()

        @pl.when(j2 < NCH // 2 - 1)
        def _():
            pltpu.async_copy(feat_hbm.at[srcw.at[e + 2]], rows0, sg0)

        scale_scatter(o, rows1)
        return 0

    lax.fori_loop(0, NCH // 2, pipe, 0)
    plsc.subcore_barrier()
    pltpu.sync_copy(out_sh.at[pl.ds(tbase, ROWS_PW)],
                    out_hbm.at[cid, pl.ds(tbase, ROWS_PW)])


_sc_agg = functools.partial(
    pl.kernel,
    out_type=jax.ShapeDtypeStruct((NC, NPAD, DH), jnp.float32),
    mesh=_mesh,
    compiler_params=_sc_params,
    scratch_types=[
        pltpu.VMEM((NCH, C), jnp.int32),
        pltpu.VMEM((NCH, C), jnp.int32),
        pltpu.VMEM((NCH, C), jnp.float32),
        pltpu.VMEM((C, DH), jnp.float32),
        pltpu.VMEM((C, DH), jnp.float32),
        pltpu.VMEM_SHARED((NPAD, DH), jnp.float32),
        pltpu.SemaphoreType.DMA,
        pltpu.SemaphoreType.DMA,
    ],
)(_sc_agg_body)


def _tc_first_body(h_ref, w_ref, al_ref, ar_ref,
                   flo_ref, fhi_ref, el_ref, er_ref):
    f = jnp.dot(h_ref[...], w_ref[...], preferred_element_type=jnp.float32)
    flo_ref[...] = f[:, :DH]
    fhi_ref[...] = f[:, DH:]
    el_ref[...] = jnp.sum(f * al_ref[...][None, :], axis=1)
    er_ref[...] = jnp.sum(f * ar_ref[...][None, :], axis=1)


_tc_out_types = [
    jax.ShapeDtypeStruct((NPAD, DH), jnp.float32),
    jax.ShapeDtypeStruct((NPAD, DH), jnp.float32),
    jax.ShapeDtypeStruct((NPAD,), jnp.float32),
    jax.ShapeDtypeStruct((NPAD,), jnp.float32),
]


def _tc_first(h, w, al, ar):
    return pl.pallas_call(_tc_first_body, out_shape=_tc_out_types)(
        h, w, al, ar)


def _tc_mid_body(plo_ref, phi_ref, b_ref, w_ref, al_ref, ar_ref,
                 flo_ref, fhi_ref, el_ref, er_ref):
    hm = jnp.concatenate(
        [plo_ref[0] + plo_ref[1], phi_ref[0] + phi_ref[1]], axis=1)
    hm = jnp.maximum(hm + b_ref[...][None, :], 0.0)
    f = jnp.dot(hm, w_ref[...], preferred_element_type=jnp.float32)
    flo_ref[...] = f[:, :DH]
    fhi_ref[...] = f[:, DH:]
    el_ref[...] = jnp.sum(f * al_ref[...][None, :], axis=1)
    er_ref[...] = jnp.sum(f * ar_ref[...][None, :], axis=1)


def _tc_mid(plo, phi, b, w, al, ar):
    return pl.pallas_call(_tc_mid_body, out_shape=_tc_out_types)(
        plo, phi, b, w, al, ar)


def _tc_final_body(plo_ref, phi_ref, b_ref, out_ref):
    hm = jnp.concatenate(
        [plo_ref[0] + plo_ref[1], phi_ref[0] + phi_ref[1]], axis=1)
    out_ref[...] = hm + b_ref[...][None, :]


def _tc_final(plo, phi, b):
    return pl.pallas_call(
        _tc_final_body,
        out_shape=jax.ShapeDtypeStruct((NPAD, D), jnp.float32),
    )(plo, phi, b)


def kernel(in_feat, edge_index, W1, al1, ar1, b1, W2, al2, ar2, b2,
           W3, al3, ar3, b3):
    src = jnp.concatenate(
        [edge_index[0], jnp.zeros((EPAD - E,), jnp.int32)]
    ).reshape(NW, NCH, C)
    dst = jnp.concatenate(
        [edge_index[1], jnp.full((EPAD - E,), N, jnp.int32)]
    ).reshape(NW, NCH, C)
    h = jnp.pad(in_feat, ((0, NPAD - N), (0, 0)))

    flo, fhi, el, er = _tc_first(h, W1, al1, ar1)
    for (b, w, al, ar) in ((b1, W2, al2, ar2), (b2, W3, al3, ar3)):
        den = _sc_denom(src, dst, el, er)
        alpha = _sc_alpha(src, dst, el, er, den)
        plo = _sc_agg(src, dst, alpha, flo)
        phi = _sc_agg(src, dst, alpha, fhi)
        flo, fhi, el, er = _tc_mid(plo, phi, b, w, al, ar)

    den = _sc_denom(src, dst, el, er)
    alpha = _sc_alpha(src, dst, el, er, den)
    plo = _sc_agg(src, dst, alpha, flo)
    phi = _sc_agg(src, dst, alpha, fhi)
    out = _tc_final(plo, phi, b3)
    return out[:N]


# speedup vs baseline: 17.4876x; 1.1883x over previous
import functools

import jax
import jax.numpy as jnp
from jax import lax
from jax.experimental import pallas as pl
from jax.experimental.pallas import tpu as pltpu
from jax.experimental.pallas import tpu_sc as plsc

N = 10000
E = 320000
D = 128

NC = 2
NS = 16
L = 16
NW = NC * NS

C = 128
EPW = 10240
NCH = EPW // C
EPAD = NW * EPW
NPAD = 10240
ROWS_PW = NPAD // NS

_mesh = plsc.VectorSubcoreMesh(
    core_axis_name="c", subcore_axis_name="s", num_cores=NC, num_subcores=NS
)


def _ew16(el_v, er_v, s16, d16):
    elg = plsc.load_gather(el_v, [s16])
    erg = plsc.load_gather(er_v, [d16])
    x = elg + erg
    e = jnp.where(x >= 0.0, x, 0.2 * x)
    return jnp.exp(e)


_sc_params = pltpu.CompilerParams(
    needs_layout_passes=False, use_tc_tiling_on_sc=False)


def _sc_denom_body(src_hbm, dst_hbm, el_hbm, er_hbm, den_out,
                   el_v, er_v, srcw, dstw, exb, zb, den_sh):
    cid = lax.axis_index("c")
    sid = lax.axis_index("s")
    wid = sid * NC + cid
    tbase = sid * ROWS_PW

    def zinit(i, _):
        zb[pl.ds(i * L, L)] = jnp.zeros((L,), jnp.float32)
        return 0

    lax.fori_loop(0, ROWS_PW // L, zinit, 0)
    pltpu.sync_copy(zb, den_sh.at[pl.ds(tbase, ROWS_PW)])
    pltpu.sync_copy(el_hbm, el_v)
    pltpu.sync_copy(er_hbm, er_v)
    pltpu.sync_copy(src_hbm.at[wid], srcw)
    pltpu.sync_copy(dst_hbm.at[wid], dstw)
    plsc.subcore_barrier()

    def chunk(j, _):
        for i in range(C // L):
            sl = pl.ds(i * L, L)
            exb[sl] = _ew16(el_v, er_v, srcw[j, sl], dstw[j, sl])
        pltpu.sync_copy(exb, den_sh.at[dstw.at[j]], add=True)
        return 0

    lax.fori_loop(0, NCH, chunk, 0)
    plsc.subcore_barrier()
    pltpu.sync_copy(den_sh.at[pl.ds(tbase, ROWS_PW)],
                    den_out.at[cid, pl.ds(tbase, ROWS_PW)])


_sc_denom = functools.partial(
    pl.kernel,
    out_type=jax.ShapeDtypeStruct((NC, NPAD), jnp.float32),
    mesh=_mesh,
    compiler_params=_sc_params,
    scratch_types=[
        pltpu.VMEM((NPAD,), jnp.float32),
        pltpu.VMEM((NPAD,), jnp.float32),
        pltpu.VMEM((NCH, C), jnp.int32),
        pltpu.VMEM((NCH, C), jnp.int32),
        pltpu.VMEM((C,), jnp.float32),
        pltpu.VMEM((ROWS_PW,), jnp.float32),
        pltpu.VMEM_SHARED((NPAD,), jnp.float32),
    ],
)(_sc_denom_body)


DH = D // 2


def _sc_agg_body(src_hbm, dst_hbm, el_hbm, er_hbm, den_hbm, feat_hbm,
                 out_hbm, el_v, er_v, den_v, tmp_v, srcw, dstw,
                 rows0, rows1, out_sh, sg0, sg1):
    cid = lax.axis_index("c")
    sid = lax.axis_index("s")
    wid = sid * NC + cid
    tbase = sid * ROWS_PW

    pltpu.sync_copy(el_hbm, el_v)
    pltpu.sync_copy(er_hbm, er_v)
    pltpu.sync_copy(den_hbm.at[0], den_v)
    pltpu.sync_copy(den_hbm.at[1], tmp_v)
    pltpu.sync_copy(src_hbm.at[wid], srcw)
    pltpu.sync_copy(dst_hbm.at[wid], dstw)

    def dsum(i, _):
        sl = pl.ds(i * L, L)
        den_v[sl] = den_v[sl] + tmp_v[sl] + 1e-9
        return 0

    lax.fori_loop(0, NPAD // L, dsum, 0)

    def zrow(r, _):
        for k in range(DH // L):
            rows0[r, pl.ds(k * L, L)] = jnp.zeros((L,), jnp.float32)
        return 0

    lax.fori_loop(0, C, zrow, 0)
    for k in range(ROWS_PW // C):
        pltpu.sync_copy(rows0, out_sh.at[pl.ds(tbase + k * C, C)])
    plsc.subcore_barrier()

    def scale_scatter(j, rows):
        def scale(g, _):
            sl16 = pl.ds(g * L, L)
            d16 = dstw[j, sl16]
            ex = _ew16(el_v, er_v, srcw[j, sl16], d16)
            dg = plsc.load_gather(den_v, [d16])
            a16 = ex / dg
            for j2 in range(L):
                r = g * L + j2
                av = jnp.full((L,), a16[j2], jnp.float32)
                for k in range(DH // L):
                    sl = pl.ds(k * L, L)
                    rows[r, sl] = rows[r, sl] * av
            return 0

        lax.fori_loop(0, C // L, scale, 0)
        pltpu.sync_copy(rows, out_sh.at[dstw.at[j]], add=True)

    pltpu.async_copy(feat_hbm.at[srcw.at[0]], rows0, sg0)

    def pipe(j2, _):
        e = 2 * j2
        o = e + 1
        pltpu.make_async_copy(feat_hbm.at[srcw.at[e]], rows0, sg0).wait()
        pltpu.async_copy(feat_hbm.at[srcw.at[o]], rows1, sg1)
        scale_scatter(e, rows0)
        pltpu.make_async_copy(feat_hbm.at[srcw.at[o]], rows1, sg1).wait()

        @pl.when(j2 < NCH // 2 - 1)
        def _():
            pltpu.async_copy(feat_hbm.at[srcw.at[e + 2]], rows0, sg0)

        scale_scatter(o, rows1)
        return 0

    lax.fori_loop(0, NCH // 2, pipe, 0)
    plsc.subcore_barrier()
    pltpu.sync_copy(out_sh.at[pl.ds(tbase, ROWS_PW)],
                    out_hbm.at[cid, pl.ds(tbase, ROWS_PW)])


_sc_agg = functools.partial(
    pl.kernel,
    out_type=jax.ShapeDtypeStruct((NC, NPAD, DH), jnp.float32),
    mesh=_mesh,
    compiler_params=_sc_params,
    scratch_types=[
        pltpu.VMEM((NPAD,), jnp.float32),
        pltpu.VMEM((NPAD,), jnp.float32),
        pltpu.VMEM((NPAD,), jnp.float32),
        pltpu.VMEM((NPAD,), jnp.float32),
        pltpu.VMEM((NCH, C), jnp.int32),
        pltpu.VMEM((NCH, C), jnp.int32),
        pltpu.VMEM((C, DH), jnp.float32),
        pltpu.VMEM((C, DH), jnp.float32),
        pltpu.VMEM_SHARED((NPAD, DH), jnp.float32),
        pltpu.SemaphoreType.DMA,
        pltpu.SemaphoreType.DMA,
    ],
)(_sc_agg_body)


def _tc_first_body(h_ref, w_ref, al_ref, ar_ref,
                   flo_ref, fhi_ref, el_ref, er_ref):
    f = jnp.dot(h_ref[...], w_ref[...], preferred_element_type=jnp.float32)
    flo_ref[...] = f[:, :DH]
    fhi_ref[...] = f[:, DH:]
    el_ref[...] = jnp.sum(f * al_ref[...][None, :], axis=1)
    er_ref[...] = jnp.sum(f * ar_ref[...][None, :], axis=1)


_tc_out_types = [
    jax.ShapeDtypeStruct((NPAD, DH), jnp.float32),
    jax.ShapeDtypeStruct((NPAD, DH), jnp.float32),
    jax.ShapeDtypeStruct((NPAD,), jnp.float32),
    jax.ShapeDtypeStruct((NPAD,), jnp.float32),
]


def _tc_first(h, w, al, ar):
    return pl.pallas_call(_tc_first_body, out_shape=_tc_out_types)(
        h, w, al, ar)


def _tc_mid_body(plo_ref, phi_ref, b_ref, w_ref, al_ref, ar_ref,
                 flo_ref, fhi_ref, el_ref, er_ref):
    hm = jnp.concatenate(
        [plo_ref[0] + plo_ref[1], phi_ref[0] + phi_ref[1]], axis=1)
    hm = jnp.maximum(hm + b_ref[...][None, :], 0.0)
    f = jnp.dot(hm, w_ref[...], preferred_element_type=jnp.float32)
    flo_ref[...] = f[:, :DH]
    fhi_ref[...] = f[:, DH:]
    el_ref[...] = jnp.sum(f * al_ref[...][None, :], axis=1)
    er_ref[...] = jnp.sum(f * ar_ref[...][None, :], axis=1)


def _tc_mid(plo, phi, b, w, al, ar):
    return pl.pallas_call(_tc_mid_body, out_shape=_tc_out_types)(
        plo, phi, b, w, al, ar)


def _tc_final_body(plo_ref, phi_ref, b_ref, out_ref):
    hm = jnp.concatenate(
        [plo_ref[0] + plo_ref[1], phi_ref[0] + phi_ref[1]], axis=1)
    out_ref[...] = hm + b_ref[...][None, :]


def _tc_final(plo, phi, b):
    return pl.pallas_call(
        _tc_final_body,
        out_shape=jax.ShapeDtypeStruct((NPAD, D), jnp.float32),
    )(plo, phi, b)


def kernel(in_feat, edge_index, W1, al1, ar1, b1, W2, al2, ar2, b2,
           W3, al3, ar3, b3):
    src = jnp.concatenate(
        [edge_index[0], jnp.zeros((EPAD - E,), jnp.int32)]
    ).reshape(NW, NCH, C)
    dst = jnp.concatenate(
        [edge_index[1], jnp.full((EPAD - E,), N, jnp.int32)]
    ).reshape(NW, NCH, C)
    h = jnp.pad(in_feat, ((0, NPAD - N), (0, 0)))

    flo, fhi, el, er = _tc_first(h, W1, al1, ar1)
    for (b, w, al, ar) in ((b1, W2, al2, ar2), (b2, W3, al3, ar3)):
        den = _sc_denom(src, dst, el, er)
        plo = _sc_agg(src, dst, el, er, den, flo)
        phi = _sc_agg(src, dst, el, er, den, fhi)
        flo, fhi, el, er = _tc_mid(plo, phi, b, w, al, ar)

    den = _sc_denom(src, dst, el, er)
    plo = _sc_agg(src, dst, el, er, den, flo)
    phi = _sc_agg(src, dst, el, er, den, fhi)
    out = _tc_final(plo, phi, b3)
    return out[:N]
